# baseline (speedup 1.0000x reference)
import functools

import jax
import jax.numpy as jnp
from jax.experimental import pallas as pl
from jax.experimental.pallas import tpu as pltpu

_BF = jnp.bfloat16


def _band(src_ref, xb_ref, *, nrows, KW):
    Cin = src_ref.shape[1]
    L = nrows - (KW - 1)
    for kw in range(KW):
        xb_ref[pl.ds(0, L), kw * Cin:(kw + 1) * Cin] = (
            src_ref[pl.ds(kw, L), :].astype(_BF))


def _conv(src_ref, xb_ref, dst_ref, w_ref, b_ref, *, H, S, KH, KW, relu):
    _band(src_ref, xb_ref, nrows=H * S, KW=KW)
    Hout = H - KH + 1
    M = Hout * S
    acc = jnp.dot(xb_ref[pl.ds(0, M), :], w_ref[0],
                  preferred_element_type=jnp.float32)
    for kh in range(1, KH):
        acc = acc + jnp.dot(xb_ref[pl.ds(kh * S, M), :], w_ref[kh],
                            preferred_element_type=jnp.float32)
    out = acc + b_ref[...]
    if relu:
        out = jnp.maximum(out, 0.0)
    dst_ref[...] = out


def _pool(src_ref, dst_ref, *, H, W, S, C):
    from jax import lax
    Ho, Wo = H // 2, W // 2
    jj = lax.broadcasted_iota(jnp.int32, (Wo, W), 0)
    kk = lax.broadcasted_iota(jnp.int32, (Wo, W), 1)
    sel_e = (kk == 2 * jj).astype(jnp.float32)
    sel_o = (kk == 2 * jj + 1).astype(jnp.float32)
    for i in range(Ho):
        r0 = src_ref[pl.ds((2 * i) * S, W), :]
        r1 = src_ref[pl.ds((2 * i + 1) * S, W), :]
        rh = jnp.maximum(r0, r1)
        pe = jnp.dot(sel_e, rh, preferred_element_type=jnp.float32)
        po = jnp.dot(sel_o, rh, preferred_element_type=jnp.float32)
        dst_ref[pl.ds(i * (S // 2), Wo), :] = jnp.maximum(pe, po)


def _body(x_ref, w1_ref, b1_ref, w2_ref, b2_ref, w3_ref, b3_ref,
          w4_ref, b4_ref, wf1_ref, bf1_ref, wf2_ref, bf2_ref, o_ref,
          xb1, a1, xb2, a2, p1, xb3, a3, xb4, a4, p2, xbf, *, dims):
    (H0, S0, H1, H2, W2, Hp1, Sp1, H3, H4, W4, Hp2, Sp2, K, Kf) = dims
    _conv(x_ref, xb1, a1, w1_ref, b1_ref, H=H0, S=S0, KH=K, KW=K, relu=False)
    _conv(a1, xb2, a2, w2_ref, b2_ref, H=H1, S=S0, KH=K, KW=K, relu=True)
    _pool(a2, p1, H=H2, W=W2, S=S0, C=a2.shape[1])
    _conv(p1, xb3, a3, w3_ref, b3_ref, H=Hp1, S=Sp1, KH=K, KW=K, relu=False)
    _conv(a3, xb4, a4, w4_ref, b4_ref, H=H3, S=Sp1, KH=K, KW=K, relu=True)
    _pool(a4, p2, H=H4, W=W4, S=Sp1, C=a4.shape[1])
    _band(p2, xbf, nrows=Hp2 * Sp2, KW=Kf)
    acc = jnp.dot(xbf[pl.ds(0, Sp2), :], wf1_ref[0],
                  preferred_element_type=jnp.float32)
    for kh in range(1, Kf):
        acc = acc + jnp.dot(xbf[pl.ds(kh * Sp2, Sp2), :], wf1_ref[kh],
                            preferred_element_type=jnp.float32)
    h = jnp.maximum(acc[0:1, :] + bf1_ref[...], 0.0)
    out = jnp.dot(h.astype(_BF), wf2_ref[...],
                  preferred_element_type=jnp.float32) + bf2_ref[...]
    o_ref[...] = out.astype(o_ref.dtype)


def _conv_w(w):
    KH, KW, Cin, Cout = w.shape
    return w.reshape(KH, KW * Cin, Cout).astype(_BF)


def kernel(x, conv1_w, conv1_b, conv2_w, conv2_b, conv3_w, conv3_b,
           conv4_w, conv4_b, fc1_w, fc1_b, fc2_w, fc2_b):
    B, Cin, H0, W0 = x.shape
    assert Cin == 1
    K = conv1_w.shape[0]
    H1, W1 = H0 - K + 1, W0 - K + 1
    H2, W2 = H1 - K + 1, W1 - K + 1
    Hp1, Wp1 = H2 // 2, W2 // 2
    H3, W3 = Hp1 - K + 1, Wp1 - K + 1
    H4, W4 = H3 - K + 1, W3 - K + 1
    Hp2, Wp2 = H4 // 2, W4 // 2
    C1, C2 = conv1_w.shape[-1], conv2_w.shape[-1]
    C3, C4 = conv3_w.shape[-1], conv4_w.shape[-1]
    hidden = fc1_w.shape[1]
    num_classes = fc2_w.shape[1]
    Kf = Hp2
    assert Hp2 == Wp2 and fc1_w.shape[0] == C4 * Hp2 * Wp2

    S0 = (W0 + 7) // 8 * 8
    Sp1, Sp2 = S0 // 2, S0 // 4
    for S, Wout in ((S0, W1), (S0, W2), (Sp1, W3), (Sp1, W4), (Sp2, 1)):
        assert S >= Wout + K - 1 or S == Sp2

    xp = jnp.pad(x.reshape(B, H0, W0), ((0, 0), (0, 0), (0, S0 - W0)))
    xp = xp.reshape(B, H0 * S0, 1)

    w1, b1 = _conv_w(conv1_w), conv1_b.reshape(1, -1)
    w2, b2 = _conv_w(conv2_w), conv2_b.reshape(1, -1)
    w3, b3 = _conv_w(conv3_w), conv3_b.reshape(1, -1)
    w4, b4 = _conv_w(conv4_w), conv4_b.reshape(1, -1)
    wf1 = (fc1_w.reshape(C4, Hp2, Wp2, hidden)
           .transpose(1, 2, 0, 3)
           .reshape(Hp2, Wp2 * C4, hidden).astype(_BF))
    bf1 = fc1_b.reshape(1, -1)
    wf2 = fc2_w.astype(_BF)
    bf2 = fc2_b.reshape(1, -1)

    dims = (H0, S0, H1, H2, W2, Hp1, Sp1, H3, H4, W4, Hp2, Sp2, K, Kf)
    f32, bf = jnp.float32, _BF
    out = pl.pallas_call(
        functools.partial(_body, dims=dims),
        out_shape=jax.ShapeDtypeStruct((B, 1, num_classes), x.dtype),
        grid=(B,),
        in_specs=[
            pl.BlockSpec((pl.Squeezed(), H0 * S0, 1), lambda b: (b, 0, 0)),
            pl.BlockSpec(w1.shape, lambda b: (0, 0, 0)),
            pl.BlockSpec(b1.shape, lambda b: (0, 0)),
            pl.BlockSpec(w2.shape, lambda b: (0, 0, 0)),
            pl.BlockSpec(b2.shape, lambda b: (0, 0)),
            pl.BlockSpec(w3.shape, lambda b: (0, 0, 0)),
            pl.BlockSpec(b3.shape, lambda b: (0, 0)),
            pl.BlockSpec(w4.shape, lambda b: (0, 0, 0)),
            pl.BlockSpec(b4.shape, lambda b: (0, 0)),
            pl.BlockSpec(wf1.shape, lambda b: (0, 0, 0)),
            pl.BlockSpec(bf1.shape, lambda b: (0, 0)),
            pl.BlockSpec(wf2.shape, lambda b: (0, 0)),
            pl.BlockSpec(bf2.shape, lambda b: (0, 0)),
        ],
        out_specs=pl.BlockSpec((pl.Squeezed(), 1, num_classes),
                               lambda b: (b, 0, 0)),
        scratch_shapes=[
            pltpu.VMEM((H0 * S0, K * 1), bf),
            pltpu.VMEM((H1 * S0, C1), f32),
            pltpu.VMEM((H1 * S0, K * C1), bf),
            pltpu.VMEM((H2 * S0, C2), f32),
            pltpu.VMEM((Hp1 * Sp1, C2), f32),
            pltpu.VMEM((Hp1 * Sp1, K * C2), bf),
            pltpu.VMEM((H3 * Sp1, C3), f32),
            pltpu.VMEM((H3 * Sp1, K * C3), bf),
            pltpu.VMEM((H4 * Sp1, C4), f32),
            pltpu.VMEM((Hp2 * Sp2, C4), f32),
            pltpu.VMEM((Hp2 * Sp2, Kf * C4), bf),
        ],
        compiler_params=pltpu.CompilerParams(
            dimension_semantics=("parallel",),
            vmem_limit_bytes=40 * 1024 * 1024,
        ),
    )(xp, w1, b1, w2, b2, w3, b3, w4, b4, wf1, bf1, wf2, bf2)
    return out.reshape(B, num_classes)


# speedup vs baseline: 1.2438x; 1.2438x over previous
import functools

import jax
import jax.numpy as jnp
from jax.experimental import pallas as pl
from jax.experimental.pallas import tpu as pltpu

_BF = jnp.bfloat16


def _band(src_ref, xb_ref, *, nrows, KW):
    L = nrows - (KW - 1)
    band = jnp.concatenate(
        [src_ref[pl.ds(kw, L), :] for kw in range(KW)], axis=1)
    xb_ref[pl.ds(0, L), :] = band.astype(_BF)


def _conv(src_ref, xb_ref, dst_ref, w_ref, b_ref, *, H, S, KH, KW, relu):
    _band(src_ref, xb_ref, nrows=H * S, KW=KW)
    Hout = H - KH + 1
    M = Hout * S
    acc = jnp.dot(xb_ref[pl.ds(0, M), :], w_ref[0],
                  preferred_element_type=jnp.float32)
    for kh in range(1, KH):
        acc = acc + jnp.dot(xb_ref[pl.ds(kh * S, M), :], w_ref[kh],
                            preferred_element_type=jnp.float32)
    out = acc + b_ref[...]
    if relu:
        out = jnp.maximum(out, 0.0)
    dst_ref[...] = out


def _pool(src_ref, dst_ref, *, H, W, S, C):
    from jax import lax
    Ho, Wo = H // 2, W // 2
    jj = lax.broadcasted_iota(jnp.int32, (Wo, W), 0)
    kk = lax.broadcasted_iota(jnp.int32, (Wo, W), 1)
    sel_e = (kk == 2 * jj).astype(jnp.float32)
    sel_o = (kk == 2 * jj + 1).astype(jnp.float32)
    for i in range(Ho):
        r0 = src_ref[pl.ds((2 * i) * S, W), :]
        r1 = src_ref[pl.ds((2 * i + 1) * S, W), :]
        rh = jnp.maximum(r0, r1)
        pe = jnp.dot(sel_e, rh, preferred_element_type=jnp.float32)
        po = jnp.dot(sel_o, rh, preferred_element_type=jnp.float32)
        dst_ref[pl.ds(i * (S // 2), Wo), :] = jnp.maximum(pe, po)


def _body(x_ref, w1_ref, b1_ref, w2_ref, b2_ref, w3_ref, b3_ref,
          w4_ref, b4_ref, wf1_ref, bf1_ref, wf2_ref, bf2_ref, o_ref,
          xb1, a1, xb2, a2, p1, xb3, a3, xb4, a4, p2, xbf, *, dims):
    (H0, S0, H1, H2, W2, Hp1, Sp1, H3, H4, W4, Hp2, Sp2, K, Kf) = dims
    _conv(x_ref, xb1, a1, w1_ref, b1_ref, H=H0, S=S0, KH=K, KW=K, relu=False)
    _conv(a1, xb2, a2, w2_ref, b2_ref, H=H1, S=S0, KH=K, KW=K, relu=True)
    _pool(a2, p1, H=H2, W=W2, S=S0, C=a2.shape[1])
    _conv(p1, xb3, a3, w3_ref, b3_ref, H=Hp1, S=Sp1, KH=K, KW=K, relu=False)
    _conv(a3, xb4, a4, w4_ref, b4_ref, H=H3, S=Sp1, KH=K, KW=K, relu=True)
    _pool(a4, p2, H=H4, W=W4, S=Sp1, C=a4.shape[1])
    _band(p2, xbf, nrows=Hp2 * Sp2, KW=Kf)
    acc = jnp.dot(xbf[pl.ds(0, Sp2), :], wf1_ref[0],
                  preferred_element_type=jnp.float32)
    for kh in range(1, Kf):
        acc = acc + jnp.dot(xbf[pl.ds(kh * Sp2, Sp2), :], wf1_ref[kh],
                            preferred_element_type=jnp.float32)
    h = jnp.maximum(acc[0:1, :] + bf1_ref[...], 0.0)
    out = jnp.dot(h.astype(_BF), wf2_ref[...],
                  preferred_element_type=jnp.float32) + bf2_ref[...]
    o_ref[...] = out.astype(o_ref.dtype)


def _conv_w(w):
    KH, KW, Cin, Cout = w.shape
    return w.reshape(KH, KW * Cin, Cout).astype(_BF)


def kernel(x, conv1_w, conv1_b, conv2_w, conv2_b, conv3_w, conv3_b,
           conv4_w, conv4_b, fc1_w, fc1_b, fc2_w, fc2_b):
    B, Cin, H0, W0 = x.shape
    assert Cin == 1
    K = conv1_w.shape[0]
    H1, W1 = H0 - K + 1, W0 - K + 1
    H2, W2 = H1 - K + 1, W1 - K + 1
    Hp1, Wp1 = H2 // 2, W2 // 2
    H3, W3 = Hp1 - K + 1, Wp1 - K + 1
    H4, W4 = H3 - K + 1, W3 - K + 1
    Hp2, Wp2 = H4 // 2, W4 // 2
    C1, C2 = conv1_w.shape[-1], conv2_w.shape[-1]
    C3, C4 = conv3_w.shape[-1], conv4_w.shape[-1]
    hidden = fc1_w.shape[1]
    num_classes = fc2_w.shape[1]
    Kf = Hp2
    assert Hp2 == Wp2 and fc1_w.shape[0] == C4 * Hp2 * Wp2

    S0 = (W0 + 7) // 8 * 8
    Sp1, Sp2 = S0 // 2, S0 // 4
    for S, Wout in ((S0, W1), (S0, W2), (Sp1, W3), (Sp1, W4), (Sp2, 1)):
        assert S >= Wout + K - 1 or S == Sp2

    xp = jnp.pad(x.reshape(B, H0, W0), ((0, 0), (0, 0), (0, S0 - W0)))
    xp = xp.reshape(B, H0 * S0, 1)

    w1, b1 = _conv_w(conv1_w), conv1_b.reshape(1, -1)
    w2, b2 = _conv_w(conv2_w), conv2_b.reshape(1, -1)
    w3, b3 = _conv_w(conv3_w), conv3_b.reshape(1, -1)
    w4, b4 = _conv_w(conv4_w), conv4_b.reshape(1, -1)
    wf1 = (fc1_w.reshape(C4, Hp2, Wp2, hidden)
           .transpose(1, 2, 0, 3)
           .reshape(Hp2, Wp2 * C4, hidden).astype(_BF))
    bf1 = fc1_b.reshape(1, -1)
    wf2 = fc2_w.astype(_BF)
    bf2 = fc2_b.reshape(1, -1)

    dims = (H0, S0, H1, H2, W2, Hp1, Sp1, H3, H4, W4, Hp2, Sp2, K, Kf)
    f32, bf = jnp.float32, _BF
    out = pl.pallas_call(
        functools.partial(_body, dims=dims),
        out_shape=jax.ShapeDtypeStruct((B, 1, num_classes), x.dtype),
        grid=(B,),
        in_specs=[
            pl.BlockSpec((pl.Squeezed(), H0 * S0, 1), lambda b: (b, 0, 0)),
            pl.BlockSpec(w1.shape, lambda b: (0, 0, 0)),
            pl.BlockSpec(b1.shape, lambda b: (0, 0)),
            pl.BlockSpec(w2.shape, lambda b: (0, 0, 0)),
            pl.BlockSpec(b2.shape, lambda b: (0, 0)),
            pl.BlockSpec(w3.shape, lambda b: (0, 0, 0)),
            pl.BlockSpec(b3.shape, lambda b: (0, 0)),
            pl.BlockSpec(w4.shape, lambda b: (0, 0, 0)),
            pl.BlockSpec(b4.shape, lambda b: (0, 0)),
            pl.BlockSpec(wf1.shape, lambda b: (0, 0, 0)),
            pl.BlockSpec(bf1.shape, lambda b: (0, 0)),
            pl.BlockSpec(wf2.shape, lambda b: (0, 0)),
            pl.BlockSpec(bf2.shape, lambda b: (0, 0)),
        ],
        out_specs=pl.BlockSpec((pl.Squeezed(), 1, num_classes),
                               lambda b: (b, 0, 0)),
        scratch_shapes=[
            pltpu.VMEM((H0 * S0, K * 1), bf),
            pltpu.VMEM((H1 * S0, C1), f32),
            pltpu.VMEM((H1 * S0, K * C1), bf),
            pltpu.VMEM((H2 * S0, C2), f32),
            pltpu.VMEM((Hp1 * Sp1, C2), f32),
            pltpu.VMEM((Hp1 * Sp1, K * C2), bf),
            pltpu.VMEM((H3 * Sp1, C3), f32),
            pltpu.VMEM((H3 * Sp1, K * C3), bf),
            pltpu.VMEM((H4 * Sp1, C4), f32),
            pltpu.VMEM((Hp2 * Sp2, C4), f32),
            pltpu.VMEM((Hp2 * Sp2, Kf * C4), bf),
        ],
        compiler_params=pltpu.CompilerParams(
            dimension_semantics=("parallel",),
            vmem_limit_bytes=40 * 1024 * 1024,
        ),
    )(xp, w1, b1, w2, b2, w3, b3, w4, b4, wf1, bf1, wf2, bf2)
    return out.reshape(B, num_classes)
